# Initial kernel scaffold; baseline (speedup 1.0000x reference)
import jax
import jax.numpy as jnp
from jax.experimental import pallas as pl


def kernel(user_emb, item_emb, adj_rows, adj_cols, adj_vals):
    raise NotImplementedError("write your pallas kernel here")


# speedup vs baseline: 2.5681x; 2.5681x over previous
import jax
import jax.numpy as jnp
from jax import lax
from jax.experimental import pallas as pl
from jax.experimental.pallas import tpu as pltpu
from jax.experimental.pallas import tpu_sc as plsc

_N_USER = 60000
_N_ITEM = 40000
_N = _N_USER + _N_ITEM
_D = 64
_NNZ = 1200000

_NS = 16
_L = 16
_HALF = _N // 2
_RPAD = 25088
_TROWS = _RPAD // _NS
_PCNT = (_RPAD, _HALF - _RPAD)
_S = 2400
_SG = _S // _L
_NB = _NNZ // _S
_MAXBLK = (_NB + _NS - 1) // _NS
_B = 128
_BG = _B // _L
_HG = _SG + _BG
_ZCH = 56


def _spmm_body(ego, rows, cols, vals, out, acc, rowb, colb, valb,
               hcol, hrow, hval, gbuf, idxg, idxr, zbuf, sem):
    c = lax.axis_index("c")
    s = lax.axis_index("s")
    lane = jax.lax.iota(jnp.int32, 16)
    zf16 = jnp.zeros((16,), jnp.float32)
    zi16 = jnp.zeros((16,), jnp.int32)

    def _zb(i, _):
        zbuf[i // 4, pl.ds((i % 4) * 16, 16)] = zf16
        return 0
    lax.fori_loop(0, _ZCH * 4, _zb, 0)

    for p in range(2):
        plo = c * _HALF + p * _RPAD
        phi = plo + _PCNT[p]

        def _zero(j, _):
            pltpu.sync_copy(zbuf, acc.at[pl.ds(s * _TROWS + j * _ZCH, _ZCH)])
            return 0
        lax.fori_loop(0, _TROWS // _ZCH, _zero, 0)
        plsc.subcore_barrier()

        def _block(k, _):
            b = s + _NS * k

            @pl.when(b < _NB)
            def _():
                g0 = b * _SG
                pltpu.sync_copy(rows.at[pl.ds(g0, _SG)], rowb)
                pltpu.sync_copy(cols.at[pl.ds(g0, _SG)], colb)
                pltpu.sync_copy(vals.at[pl.ds(g0, _SG)], valb)

                def _scan(i, cnt):
                    r = rowb[i, :]
                    msk = (r >= plo) & (r < phi)
                    pos = plsc.cumsum(msk.astype(jnp.int32))
                    idx = cnt + pos - 1
                    ir = lax.shift_right_logical(idx, 4)
                    il = lax.bitwise_and(idx, 15)
                    plsc.store_scatter(hcol, [ir, il], colb[i, :], mask=msk)
                    plsc.store_scatter(hval, [ir, il], valb[i, :], mask=msk)
                    plsc.store_scatter(hrow, [ir, il], r - plo, mask=msk)
                    return cnt + jnp.max(pos)
                hits = lax.fori_loop(0, _SG, _scan, jnp.int32(0))

                ones = msk_true = lane >= 0
                for j in range(_BG):
                    idx = hits + j * 16 + lane
                    ir = lax.shift_right_logical(idx, 4)
                    il = lax.bitwise_and(idx, 15)
                    plsc.store_scatter(hcol, [ir, il], zi16, mask=msk_true)
                    plsc.store_scatter(hval, [ir, il], zf16, mask=msk_true)
                    plsc.store_scatter(hrow, [ir, il], zi16, mask=msk_true)

                nbat = (hits + _B - 1) // _B

                def _batch(bi, _):
                    bg = bi * _BG
                    for q in range(_BG):
                        idxg[pl.ds(q * 16, 16)] = hcol[bg + q, :]
                        idxr[pl.ds(q * 16, 16)] = hrow[bg + q, :]
                    pltpu.async_copy(ego.at[idxg], gbuf, sem).wait()

                    def _scale(g, _):
                        vv = hval[bg + g, :]
                        for e in range(16):
                            row = g * 16 + e
                            v = vv[e]
                            for q in range(4):
                                gbuf[row, pl.ds(q * 16, 16)] = (
                                    gbuf[row, pl.ds(q * 16, 16)] * v)
                        return 0
                    lax.fori_loop(0, _BG, _scale, 0)

                    pltpu.sync_copy(gbuf, acc.at[idxr], add=True)
                    return 0
                lax.fori_loop(0, nbat, _batch, 0)
            return 0
        lax.fori_loop(0, _MAXBLK, _block, 0)
        plsc.subcore_barrier()

        def _wout(j, _):
            rb = s * _TROWS + j * 16

            @pl.when(rb < _PCNT[p])
            def _():
                pltpu.sync_copy(acc.at[pl.ds(rb, 16)],
                                out.at[pl.ds(plo + rb, 16)])
            return 0
        lax.fori_loop(0, _TROWS // 16, _wout, 0)
        plsc.subcore_barrier()


def _spmm(ego, rows2, cols2, vals2):
    mesh = plsc.VectorSubcoreMesh(core_axis_name="c", subcore_axis_name="s")
    return pl.kernel(
        _spmm_body,
        out_type=jax.ShapeDtypeStruct((_N, _D), jnp.float32),
        mesh=mesh,
        compiler_params=pltpu.CompilerParams(
            needs_layout_passes=False, use_tc_tiling_on_sc=False),
        scratch_types=[
            pltpu.VMEM_SHARED((_RPAD, _D), jnp.float32),
            pltpu.VMEM((_SG, _L), jnp.int32),
            pltpu.VMEM((_SG, _L), jnp.int32),
            pltpu.VMEM((_SG, _L), jnp.float32),
            pltpu.VMEM((_HG, _L), jnp.int32),
            pltpu.VMEM((_HG, _L), jnp.int32),
            pltpu.VMEM((_HG, _L), jnp.float32),
            pltpu.VMEM((_B, _D), jnp.float32),
            pltpu.VMEM((_B,), jnp.int32),
            pltpu.VMEM((_B,), jnp.int32),
            pltpu.VMEM((_ZCH, _D), jnp.float32),
            pltpu.SemaphoreType.DMA,
        ],
    )(ego, rows2, cols2, vals2)


def _mean3(e0_ref, e1_ref, e2_ref, o_ref):
    o_ref[...] = (e0_ref[...] + e1_ref[...] + e2_ref[...]) * (1.0 / 3.0)


def _combine(ego0, ego1, ego2, nrows, row_off):
    blk = 2000
    off = row_off // blk
    return pl.pallas_call(
        _mean3,
        grid=(nrows // blk,),
        in_specs=[pl.BlockSpec((blk, _D), lambda i: (i + off, 0))] * 3,
        out_specs=pl.BlockSpec((blk, _D), lambda i: (i, 0)),
        out_shape=jax.ShapeDtypeStruct((nrows, _D), jnp.float32),
    )(ego0, ego1, ego2)


def kernel(user_emb, item_emb, adj_rows, adj_cols, adj_vals):
    ego0 = jnp.concatenate([user_emb, item_emb], axis=0)
    rows2 = adj_rows.reshape(_NNZ // _L, _L)
    cols2 = adj_cols.reshape(_NNZ // _L, _L)
    vals2 = adj_vals.reshape(_NNZ // _L, _L)
    ego1 = _spmm(ego0, rows2, cols2, vals2)
    ego2 = _spmm(ego1, rows2, cols2, vals2)
    user = _combine(ego0, ego1, ego2, _N_USER, 0)
    item = _combine(ego0, ego1, ego2, _N_ITEM, _N_USER)
    return (user, item)


# speedup vs baseline: 3.2996x; 1.2848x over previous
import jax
import jax.numpy as jnp
from jax import lax
from jax.experimental import pallas as pl
from jax.experimental.pallas import tpu as pltpu
from jax.experimental.pallas import tpu_sc as plsc

_N_USER = 60000
_N_ITEM = 40000
_N = _N_USER + _N_ITEM
_D = 64
_NNZ = 1200000

_NS = 16
_L = 16
_HALF = _N // 2
_RPAD = 16896
_TROWS = _RPAD // _NS
_OFFS = (0, 16208, 33104)
_S = 6000
_SG = _S // _L
_NB = _NNZ // _S
_MAXBLK = (_NB + _NS - 1) // _NS
_B = 128
_BG = _B // _L
_HG = _SG + _BG
_ZCH = 132


def _spmm_body(ego, rows, cols, vals, out, acc, rowb, colb, valb,
               hcol, hrow, hval, gbufa, gbufb, idxga, idxgb, idxra, idxrb,
               zbuf, seme, sema, semb):
    c = lax.axis_index("c")
    s = lax.axis_index("s")
    lane = jax.lax.iota(jnp.int32, 16)
    zf16 = jnp.zeros((16,), jnp.float32)
    zi16 = jnp.zeros((16,), jnp.int32)
    bufs = ((gbufa, idxga, idxra, sema), (gbufb, idxgb, idxrb, semb))

    def _zb(i, _):
        zbuf[i // 4, pl.ds((i % 4) * 16, 16)] = zf16
        return 0
    lax.fori_loop(0, _ZCH * 4, _zb, 0)

    for p in range(3):
        plo = c * _HALF + _OFFS[p]
        phi = plo + _RPAD

        def _zero(j, _):
            pltpu.sync_copy(zbuf, acc.at[pl.ds(s * _TROWS + j * _ZCH, _ZCH)])
            return 0
        lax.fori_loop(0, _TROWS // _ZCH, _zero, 0)
        plsc.subcore_barrier()

        def _block(k, _):
            b = s + _NS * k

            @pl.when(b < _NB)
            def _():
                g0 = b * _SG
                de_r = pltpu.async_copy(rows.at[pl.ds(g0, _SG)], rowb, seme)
                de_c = pltpu.async_copy(cols.at[pl.ds(g0, _SG)], colb, seme)
                de_v = pltpu.async_copy(vals.at[pl.ds(g0, _SG)], valb, seme)
                de_r.wait()
                de_c.wait()
                de_v.wait()

                def _scan(i, cnt):
                    r = rowb[i, :]
                    msk = (r >= plo) & (r < phi)
                    pos = plsc.cumsum(msk.astype(jnp.int32))
                    idx = cnt + pos - 1
                    ir = lax.shift_right_logical(idx, 4)
                    il = lax.bitwise_and(idx, 15)
                    plsc.store_scatter(hcol, [ir, il], colb[i, :], mask=msk)
                    plsc.store_scatter(hval, [ir, il], valb[i, :], mask=msk)
                    plsc.store_scatter(hrow, [ir, il], r - plo, mask=msk)
                    return cnt + jnp.max(pos)
                hits = lax.fori_loop(0, _SG, _scan, jnp.int32(0))

                msk_true = lane >= 0
                for j in range(_BG):
                    idx = hits + j * 16 + lane
                    ir = lax.shift_right_logical(idx, 4)
                    il = lax.bitwise_and(idx, 15)
                    plsc.store_scatter(hcol, [ir, il], zi16, mask=msk_true)
                    plsc.store_scatter(hval, [ir, il], zf16, mask=msk_true)
                    plsc.store_scatter(hrow, [ir, il], zi16, mask=msk_true)

                nbat = (hits + _B - 1) // _B

                def _stage_and_fire(bi, buf):
                    gbuf, idxg, idxr, sem = buf
                    bg = bi * _BG
                    for q in range(_BG):
                        idxg[pl.ds(q * 16, 16)] = hcol[bg + q, :]
                        idxr[pl.ds(q * 16, 16)] = hrow[bg + q, :]
                    pltpu.async_copy(ego.at[idxg], gbuf, sem)

                @pl.when(nbat > 0)
                def _():
                    _stage_and_fire(jnp.int32(0), bufs[0])

                def _pair(pi, _):
                    for x in range(2):
                        bi = pi * 2 + x
                        gbuf, idxg, idxr, sem = bufs[x]

                        @pl.when(bi < nbat)
                        def _():
                            @pl.when(bi + 1 < nbat)
                            def _():
                                _stage_and_fire(bi + 1, bufs[1 - x])
                            pltpu.make_async_copy(
                                ego.at[idxg], gbuf, sem).wait()

                            bg = bi * _BG

                            def _scale(g, _):
                                vv = hval[bg + g, :]
                                for e in range(16):
                                    row = g * 16 + e
                                    v = vv[e]
                                    for q in range(4):
                                        gbuf[row, pl.ds(q * 16, 16)] = (
                                            gbuf[row, pl.ds(q * 16, 16)] * v)
                                return 0
                            lax.fori_loop(0, _BG, _scale, 0)

                            pltpu.sync_copy(gbuf, acc.at[idxr], add=True)
                    return 0
                lax.fori_loop(0, (nbat + 1) // 2, _pair, 0)
            return 0
        lax.fori_loop(0, _MAXBLK, _block, 0)
        plsc.subcore_barrier()

        pltpu.sync_copy(acc.at[pl.ds(s * _TROWS, _TROWS)],
                        out.at[pl.ds(plo + s * _TROWS, _TROWS)])
        plsc.subcore_barrier()


def _spmm(ego, rows2, cols2, vals2):
    mesh = plsc.VectorSubcoreMesh(core_axis_name="c", subcore_axis_name="s")
    return pl.kernel(
        _spmm_body,
        out_type=jax.ShapeDtypeStruct((_N, _D), jnp.float32),
        mesh=mesh,
        compiler_params=pltpu.CompilerParams(
            needs_layout_passes=False, use_tc_tiling_on_sc=False),
        scratch_types=[
            pltpu.VMEM_SHARED((_RPAD, _D), jnp.float32),
            pltpu.VMEM((_SG, _L), jnp.int32),
            pltpu.VMEM((_SG, _L), jnp.int32),
            pltpu.VMEM((_SG, _L), jnp.float32),
            pltpu.VMEM((_HG, _L), jnp.int32),
            pltpu.VMEM((_HG, _L), jnp.int32),
            pltpu.VMEM((_HG, _L), jnp.float32),
            pltpu.VMEM((_B, _D), jnp.float32),
            pltpu.VMEM((_B, _D), jnp.float32),
            pltpu.VMEM((_B,), jnp.int32),
            pltpu.VMEM((_B,), jnp.int32),
            pltpu.VMEM((_B,), jnp.int32),
            pltpu.VMEM((_B,), jnp.int32),
            pltpu.VMEM((_ZCH, _D), jnp.float32),
            pltpu.SemaphoreType.DMA,
            pltpu.SemaphoreType.DMA,
            pltpu.SemaphoreType.DMA,
        ],
    )(ego, rows2, cols2, vals2)


def _mean3(e0_ref, e1_ref, e2_ref, o_ref):
    o_ref[...] = (e0_ref[...] + e1_ref[...] + e2_ref[...]) * (1.0 / 3.0)


def _combine(ego0, ego1, ego2, nrows, row_off):
    blk = 2000
    off = row_off // blk
    return pl.pallas_call(
        _mean3,
        grid=(nrows // blk,),
        in_specs=[pl.BlockSpec((blk, _D), lambda i: (i + off, 0))] * 3,
        out_specs=pl.BlockSpec((blk, _D), lambda i: (i, 0)),
        out_shape=jax.ShapeDtypeStruct((nrows, _D), jnp.float32),
    )(ego0, ego1, ego2)


def kernel(user_emb, item_emb, adj_rows, adj_cols, adj_vals):
    ego0 = jnp.concatenate([user_emb, item_emb], axis=0)
    rows2 = adj_rows.reshape(_NNZ // _L, _L)
    cols2 = adj_cols.reshape(_NNZ // _L, _L)
    vals2 = adj_vals.reshape(_NNZ // _L, _L)
    ego1 = _spmm(ego0, rows2, cols2, vals2)
    ego2 = _spmm(ego1, rows2, cols2, vals2)
    user = _combine(ego0, ego1, ego2, _N_USER, 0)
    item = _combine(ego0, ego1, ego2, _N_ITEM, _N_USER)
    return (user, item)
